# baseline (speedup 1.0000x reference)
import jax
import jax.numpy as jnp
import numpy as np
from jax import lax
from jax.experimental import pallas as pl
from jax.experimental.pallas import tpu as pltpu
from jax.experimental.pallas import tpu_sc as plsc

_MAX_REL = 7
_NUM_BUCKETS = 2 * _MAX_REL + 1
_NUM_HEADS = 16
_NUM_TILES = 32
_IDX_LEN = 128
_ZERO_ROW = _NUM_BUCKETS * _NUM_BUCKETS


def _flat_idx(i: int, j: int) -> int:
    dr = i // 8 - j // 8
    df = i % 8 - j % 8
    return (dr + _MAX_REL) * _NUM_BUCKETS + (df + _MAX_REL)


def _host_indices() -> np.ndarray:
    idx = np.full((_NUM_TILES, _IDX_LEN), _ZERO_ROW, np.int32)
    for w in range(_NUM_TILES):
        for li in range(2):
            i = 2 * w + li
            for c in range(1, 64):
                idx[w, li * 64 + c] = _flat_idx(i, c - 1)
    return idx


_IDX_HOST = _host_indices()


def _tanh2(g):
    g = jnp.clip(g, -20.0, 20.0)
    e = jnp.exp(g * -2.0)
    return (2.0 - 2.0 * e) / (1.0 + e)


def _body(table_hbm, idx_hbm, out_hbm, table_v, idx_v, buf):
    wid = lax.axis_index("s") * 2 + lax.axis_index("c")
    pltpu.sync_copy(table_hbm, table_v.at[:_ZERO_ROW])
    pltpu.sync_copy(idx_hbm.at[wid], idx_v)
    table_v[_ZERO_ROW, :] = jnp.zeros((16,), jnp.float32)

    lane = lax.iota(jnp.int32, 16)
    zeros = jnp.zeros((16,), jnp.float32)

    for k in range(8):
        li = k // 4
        c0 = (k % 4) * 16
        ivec = idx_v[pl.ds(k * 16, 16)]
        for h in range(_NUM_HEADS):
            g = plsc.load_gather(table_v, [ivec, jnp.full((16,), h, jnp.int32)])
            buf[h, 1 + li, pl.ds(c0, 16)] = _tanh2(g)

    for li in range(2):
        i = 2 * wid + li
        tail = (i // 8) * _NUM_BUCKETS + lax.rem(i, 8)
        row = table_v[tail, :]
        plsc.store_scatter(
            buf, [lane, jnp.full((16,), 1 + li, jnp.int32),
                  jnp.full((16,), 64, jnp.int32)], _tanh2(row))

    @pl.when(wid == 0)
    def _():
        for h in range(_NUM_HEADS):
            for c0 in range(0, 64, 16):
                buf[h, 0, pl.ds(c0, 16)] = zeros
        plsc.store_scatter(
            buf, [lane, jnp.zeros((16,), jnp.int32),
                  jnp.full((16,), 64, jnp.int32)], zeros)
        pltpu.sync_copy(buf, out_hbm.at[:, pl.ds(0, 3), :])

    @pl.when(wid != 0)
    def _():
        pltpu.sync_copy(buf.at[:, 1:, :],
                        out_hbm.at[:, pl.ds(2 * wid + 1, 2), :])


@jax.jit
def _run(table2d):
    mesh = plsc.VectorSubcoreMesh(core_axis_name="c", subcore_axis_name="s")
    out = pl.kernel(
        _body,
        out_type=jax.ShapeDtypeStruct((_NUM_HEADS, 65, 65), jnp.float32),
        mesh=mesh,
        compiler_params=pltpu.CompilerParams(use_tc_tiling_on_sc=False,
                                             needs_layout_passes=False),
        scratch_types=[
            pltpu.VMEM((_ZERO_ROW + 1, _NUM_HEADS), jnp.float32),
            pltpu.VMEM((_IDX_LEN,), jnp.int32),
            pltpu.VMEM((_NUM_HEADS, 3, 65), jnp.float32),
        ],
    )(table2d, jnp.asarray(_IDX_HOST))
    return out.reshape(1, _NUM_HEADS, 65, 65)


def kernel(relative_bias_table, seq_len):
    del seq_len
    return _run(relative_bias_table.reshape(_ZERO_ROW, _NUM_HEADS))
